# baseline (speedup 1.0000x reference)
import functools

import jax
import jax.numpy as jnp
from jax import lax
from jax.experimental import pallas as pl
from jax.experimental.pallas import tpu as pltpu
from jax.experimental.pallas import tpu_sc as plsc

N = 160000
K = 16
NC, NS = 2, 16
NW = NC * NS
ROWS_W = N // NW
CH = 1000
NCH = ROWS_W // CH
CHA = 200
NCHA = ROWS_W // CHA
_GSLICES = [(o, min(128, CHA - o)) for o in range(0, CHA, 128)]

PCB = 40
NPIECE = CH // PCB


def _sc_a(dist_hbm, angle_hbm, idxh_hbm, idxt_hbm, icls_hbm,
          dtf_out, atf_out, icf_out, mm_out,
          idx_v, cidx_v, icls_v, drows_v, arows_v,
          dout_v, aout_v, iout_v, mm_v, sems, semo):
    wid = lax.axis_index("s") * NC + lax.axis_index("c")
    base_w = wid * ROWS_W

    def stage(ch, b):
        base = base_w + ch * CHA
        pltpu.sync_copy(idxh_hbm.at[pl.ds(base, CHA)], idx_v.at[b])
        cps = [
            pltpu.async_copy(idxt_hbm.at[pl.ds(base * K, CHA * K)],
                             cidx_v.at[b], sems.at[b]),
            pltpu.async_copy(icls_hbm.at[pl.ds(base * K, CHA * K)],
                             icls_v.at[b], sems.at[b]),
        ]
        for off, sz in _GSLICES:
            sl = pl.ds(off, sz)
            cps.append(pltpu.async_copy(
                dist_hbm.at[idx_v.at[b].at[sl]],
                drows_v.at[b].at[sl, :], sems.at[b]))
            cps.append(pltpu.async_copy(
                angle_hbm.at[idx_v.at[b].at[sl]],
                arows_v.at[b].at[sl, :], sems.at[b]))
        return cps

    def compute(b, carry):
        def row_body(i, rc):
            rmn, rmx = rc
            colraw = cidx_v[b, pl.ds(i * K, K)]
            col = jnp.minimum(colraw, K - 1)
            row = jnp.full((16,), i, jnp.int32)
            msk = colraw < K
            d = plsc.load_gather(drows_v.at[b], [row, col])
            a = plsc.load_gather(arows_v.at[b], [row, col])
            d = jnp.where(msk, d, 0.0)
            a = jnp.where(msk, a, 0.0)
            fl = pl.ds(i * K, K)
            dout_v[b, fl] = d
            aout_v[b, fl] = a
            iout_v[b, fl] = icls_v[b, fl]
            return jnp.minimum(rmn, d), jnp.maximum(rmx, d)

        return lax.fori_loop(0, CHA, row_body, carry)

    mn = jnp.full((16,), jnp.inf, jnp.float32)
    mx = jnp.full((16,), -jnp.inf, jnp.float32)
    in_cps = {0: stage(0, 0), 1: stage(1, 1)}
    out_cps = {}
    for ch in range(NCHA):
        b = ch % 2
        for c in in_cps.pop(ch):
            c.wait()
        if ch >= 2:
            for c in out_cps.pop(ch - 2):
                c.wait()
        mn, mx = compute(b, (mn, mx))
        fbase = (base_w + ch * CHA) * K
        out_cps[ch] = [
            pltpu.async_copy(dout_v.at[b],
                             dtf_out.at[pl.ds(fbase, CHA * K)], semo.at[b]),
            pltpu.async_copy(aout_v.at[b],
                             atf_out.at[pl.ds(fbase, CHA * K)], semo.at[b]),
            pltpu.async_copy(iout_v.at[b],
                             icf_out.at[pl.ds(fbase, CHA * K)], semo.at[b]),
        ]
        if ch + 2 < NCHA:
            in_cps[ch + 2] = stage(ch + 2, b)
    for ch in (NCHA - 2, NCHA - 1):
        for c in out_cps.pop(ch):
            c.wait()

    mm_v[pl.ds(0, 16)] = mn
    mm_v[pl.ds(16, 16)] = mx
    pltpu.sync_copy(mm_v.at[pl.ds(0, 16)], mm_out.at[pl.ds(wid * 16, 16)])
    pltpu.sync_copy(mm_v.at[pl.ds(16, 16)],
                    mm_out.at[pl.ds(NW * 16 + wid * 16, 16)])


@functools.lru_cache(maxsize=1)
def _sc_a_call():
    return functools.partial(
        pl.kernel,
        out_type=[
            jax.ShapeDtypeStruct((N * K,), jnp.float32),
            jax.ShapeDtypeStruct((N * K,), jnp.float32),
            jax.ShapeDtypeStruct((N * K,), jnp.int32),
            jax.ShapeDtypeStruct((2 * NW * 16,), jnp.float32),
        ],
        mesh=plsc.VectorSubcoreMesh(
            core_axis_name="c", subcore_axis_name="s",
            num_cores=NC, num_subcores=NS),
        compiler_params=pltpu.CompilerParams(
            needs_layout_passes=False, use_tc_tiling_on_sc=False),
        scratch_types=[
            pltpu.VMEM((2, CHA), jnp.int32),
            pltpu.VMEM((2, CHA * K), jnp.int32),
            pltpu.VMEM((2, CHA * K), jnp.int32),
            pltpu.VMEM((2, CHA, K), jnp.float32),
            pltpu.VMEM((2, CHA, K), jnp.float32),
            pltpu.VMEM((2, CHA * K), jnp.float32),
            pltpu.VMEM((2, CHA * K), jnp.float32),
            pltpu.VMEM((2, CHA * K), jnp.int32),
            pltpu.VMEM((2 * 16,), jnp.float32),
            pltpu.SemaphoreType.DMA((2,)),
            pltpu.SemaphoreType.DMA((2,)),
        ],
    )(_sc_a)


def _sc_b(dtf_hbm, atf_hbm, icf_hbm, mm_hbm, out_hbm,
          dt_v, at_v, ic_v, mm_v, buf_v, sems, semst):
    wid = lax.axis_index("s") * NC + lax.axis_index("c")
    base_w = wid * ROWS_W

    pltpu.sync_copy(mm_hbm, mm_v)

    def red_body(w, c):
        lo, hi = c
        lo = jnp.minimum(lo, mm_v[pl.ds(w * 16, 16)])
        hi = jnp.maximum(hi, mm_v[pl.ds(NW * 16 + w * 16, 16)])
        return lo, hi

    lo0 = jnp.full((16,), jnp.inf, jnp.float32)
    hi0 = jnp.full((16,), -jnp.inf, jnp.float32)
    lo, hi = lax.fori_loop(0, NW, red_body, (lo0, hi0))
    c22 = lax.broadcasted_iota(jnp.int32, (16,), 0) * 22
    ones = jnp.full((16,), 1.0, jnp.float32)
    zeros = jnp.zeros((16,), jnp.float32)
    gminv = jnp.full((16,), jnp.min(lo), jnp.float32)
    gmaxv = jnp.full((16,), jnp.max(hi), jnp.float32)
    invv = ones / (gmaxv - gminv)

    def z_body(i, _):
        for b in range(2):
            for t in range(24):
                buf_v[b, i, pl.ds(t * 16, 16)] = zeros
        return 0

    lax.fori_loop(0, PCB, z_body, 0)

    def stage(ch, sb):
        fbase = (base_w + ch * CH) * K
        return [
            pltpu.async_copy(dtf_hbm.at[pl.ds(fbase, CH * K)],
                             dt_v.at[sb], semst.at[sb]),
            pltpu.async_copy(atf_hbm.at[pl.ds(fbase, CH * K)],
                             at_v.at[sb], semst.at[sb]),
            pltpu.async_copy(icf_hbm.at[pl.ds(fbase, CH * K)],
                             ic_v.at[sb], semst.at[sb]),
        ]

    st_cps = {0: stage(0, 0)}
    for ch in range(NCH):
        sb = ch % 2
        base = base_w + ch * CH
        for c in st_cps.pop(ch):
            c.wait()
        if ch + 1 < NCH:
            st_cps[ch + 1] = stage(ch + 1, (ch + 1) % 2)

        def restore(p, b, sb=sb):
            def row_z(i, _):
                ic = p * PCB + i
                cls = ic_v[sb, pl.ds(ic * K, K)]
                rowi = jnp.full((16,), i, jnp.int32)
                plsc.store_scatter(buf_v.at[b], [rowi, c22 + cls], zeros)
                return 0

            lax.fori_loop(0, PCB, row_z, 0)

        def piece_body(p, _, base=base, sb=sb, restore=restore):
            b = lax.rem(p, 2)

            @pl.when(p >= 2)
            def _():
                pltpu.make_async_copy(
                    buf_v.at[b],
                    out_hbm.at[pl.ds(base + p * PCB, PCB), :],
                    sems.at[b]).wait()
                restore(p - 2, b)

            def row_w(i, _):
                ic = p * PCB + i
                fl = pl.ds(ic * K, K)
                cls = ic_v[sb, fl]
                rowi = jnp.full((16,), i, jnp.int32)
                plsc.store_scatter(buf_v.at[b], [rowi, c22 + cls], ones)
                dn = (dt_v[sb, fl] - gminv) * invv
                buf_v[b, i, pl.ds(352, 16)] = dn
                buf_v[b, i, pl.ds(368, 16)] = at_v[sb, fl]
                return 0

            lax.fori_loop(0, PCB, row_w, 0)
            pltpu.async_copy(
                buf_v.at[b], out_hbm.at[pl.ds(base + p * PCB, PCB), :],
                sems.at[b])
            return 0

        lax.fori_loop(0, NPIECE, piece_body, 0)
        for pp in (NPIECE - 2, NPIECE - 1):
            b = pp % 2
            pltpu.make_async_copy(
                buf_v.at[b],
                out_hbm.at[pl.ds(base + pp * PCB, PCB), :],
                sems.at[b]).wait()
            restore(pp, b)


@functools.lru_cache(maxsize=1)
def _sc_b_call():
    return functools.partial(
        pl.kernel,
        out_type=[
            jax.ShapeDtypeStruct((N, 384), jnp.float32),
        ],
        mesh=plsc.VectorSubcoreMesh(
            core_axis_name="c", subcore_axis_name="s",
            num_cores=NC, num_subcores=NS),
        compiler_params=pltpu.CompilerParams(
            needs_layout_passes=False, use_tc_tiling_on_sc=True),
        scratch_types=[
            pltpu.VMEM((2, CH * K), jnp.float32),
            pltpu.VMEM((2, CH * K), jnp.float32),
            pltpu.VMEM((2, CH * K), jnp.int32),
            pltpu.VMEM((2 * NW * 16,), jnp.float32),
            pltpu.VMEM((2, PCB, 384), jnp.float32),
            pltpu.SemaphoreType.DMA((2,)),
            pltpu.SemaphoreType.DMA((2,)),
        ],
    )(_sc_b)


def kernel(dist, angle, idx_t, index_t, index_h, device):
    del device
    idx_t = jnp.ravel(idx_t.astype(jnp.int32))
    index_t = jnp.ravel(index_t.astype(jnp.int32))
    index_h = index_h.astype(jnp.int32)

    dtf, atf, icf, mm = _sc_a_call()(dist, angle, index_h, index_t, idx_t)
    (out,) = _sc_b_call()(dtf, atf, icf, mm)
    return out


# speedup vs baseline: 1.0639x; 1.0639x over previous
import functools

import jax
import jax.numpy as jnp
from jax import lax
from jax.experimental import pallas as pl
from jax.experimental.pallas import tpu as pltpu
from jax.experimental.pallas import tpu_sc as plsc

N = 160000
K = 16
NC, NS = 2, 16
NW = NC * NS
ROWS_W = N // NW
CH = 1000
NCH = ROWS_W // CH
CHA = 200
NCHA = ROWS_W // CHA
_GSLICES = [(o, min(128, CHA - o)) for o in range(0, CHA, 128)]

PCB = 40
NPIECE = CH // PCB


CH0 = 200
NCH0 = ROWS_W // CH0


def _sc_a0(idxt_hbm, icls_hbm, idxtf_out, iclsf_out,
           t_v, c_v, tf_v, cf_v, sems, semo):
    wid = lax.axis_index("s") * NC + lax.axis_index("c")
    base_w = wid * ROWS_W

    def stage(ch, b):
        base = base_w + ch * CH0
        return [
            pltpu.async_copy(idxt_hbm.at[pl.ds(base, CH0), :],
                             t_v.at[b], sems.at[b]),
            pltpu.async_copy(icls_hbm.at[pl.ds(base, CH0), :],
                             c_v.at[b], sems.at[b]),
        ]

    in_cps = {0: stage(0, 0), 1: stage(1, 1)}
    out_cps = {}
    for ch in range(NCH0):
        b = ch % 2
        for c in in_cps.pop(ch):
            c.wait()
        if ch >= 2:
            for c in out_cps.pop(ch - 2):
                c.wait()

        def row_body(i, _, b=b):
            fl = pl.ds(i * K, K)
            tf_v[b, fl] = t_v[b, i, :]
            cf_v[b, fl] = c_v[b, i, :]
            return 0

        lax.fori_loop(0, CH0, row_body, 0)
        fbase = (base_w + ch * CH0) * K
        out_cps[ch] = [
            pltpu.async_copy(tf_v.at[b],
                             idxtf_out.at[pl.ds(fbase, CH0 * K)], semo.at[b]),
            pltpu.async_copy(cf_v.at[b],
                             iclsf_out.at[pl.ds(fbase, CH0 * K)], semo.at[b]),
        ]
        if ch + 2 < NCH0:
            in_cps[ch + 2] = stage(ch + 2, b)
    for ch in (NCH0 - 2, NCH0 - 1):
        for c in out_cps.pop(ch):
            c.wait()


@functools.lru_cache(maxsize=1)
def _sc_a0_call():
    return functools.partial(
        pl.kernel,
        out_type=[
            jax.ShapeDtypeStruct((N * K,), jnp.int32),
            jax.ShapeDtypeStruct((N * K,), jnp.int32),
        ],
        mesh=plsc.VectorSubcoreMesh(
            core_axis_name="c", subcore_axis_name="s",
            num_cores=NC, num_subcores=NS),
        compiler_params=pltpu.CompilerParams(
            needs_layout_passes=False, use_tc_tiling_on_sc=True),
        scratch_types=[
            pltpu.VMEM((2, CH0, K), jnp.int32),
            pltpu.VMEM((2, CH0, K), jnp.int32),
            pltpu.VMEM((2, CH0 * K), jnp.int32),
            pltpu.VMEM((2, CH0 * K), jnp.int32),
            pltpu.SemaphoreType.DMA((2,)),
            pltpu.SemaphoreType.DMA((2,)),
        ],
    )(_sc_a0)


def _sc_a(dist_hbm, angle_hbm, idxh_hbm, idxt_hbm,
          dtf_out, atf_out, mm_out,
          idx_v, cidx_v, drows_v, arows_v,
          dout_v, aout_v, mm_v, sems, semo):
    wid = lax.axis_index("s") * NC + lax.axis_index("c")
    base_w = wid * ROWS_W

    def stage(ch, b):
        base = base_w + ch * CHA
        pltpu.sync_copy(idxh_hbm.at[pl.ds(base, CHA)], idx_v.at[b])
        cps = [
            pltpu.async_copy(idxt_hbm.at[pl.ds(base * K, CHA * K)],
                             cidx_v.at[b], sems.at[b]),
        ]
        for off, sz in _GSLICES:
            sl = pl.ds(off, sz)
            cps.append(pltpu.async_copy(
                dist_hbm.at[idx_v.at[b].at[sl]],
                drows_v.at[b].at[sl, :], sems.at[b]))
            cps.append(pltpu.async_copy(
                angle_hbm.at[idx_v.at[b].at[sl]],
                arows_v.at[b].at[sl, :], sems.at[b]))
        return cps

    def compute(b, carry):
        def row_body(i, rc):
            rmn, rmx = rc
            colraw = cidx_v[b, pl.ds(i * K, K)]
            col = jnp.minimum(colraw, K - 1)
            row = jnp.full((16,), i, jnp.int32)
            msk = colraw < K
            d = plsc.load_gather(drows_v.at[b], [row, col])
            a = plsc.load_gather(arows_v.at[b], [row, col])
            d = jnp.where(msk, d, 0.0)
            a = jnp.where(msk, a, 0.0)
            fl = pl.ds(i * K, K)
            dout_v[b, fl] = d
            aout_v[b, fl] = a
            return jnp.minimum(rmn, d), jnp.maximum(rmx, d)

        return lax.fori_loop(0, CHA, row_body, carry)

    mn = jnp.full((16,), jnp.inf, jnp.float32)
    mx = jnp.full((16,), -jnp.inf, jnp.float32)
    in_cps = {0: stage(0, 0), 1: stage(1, 1)}
    out_cps = {}
    for ch in range(NCHA):
        b = ch % 2
        for c in in_cps.pop(ch):
            c.wait()
        if ch >= 2:
            for c in out_cps.pop(ch - 2):
                c.wait()
        mn, mx = compute(b, (mn, mx))
        fbase = (base_w + ch * CHA) * K
        out_cps[ch] = [
            pltpu.async_copy(dout_v.at[b],
                             dtf_out.at[pl.ds(fbase, CHA * K)], semo.at[b]),
            pltpu.async_copy(aout_v.at[b],
                             atf_out.at[pl.ds(fbase, CHA * K)], semo.at[b]),
        ]
        if ch + 2 < NCHA:
            in_cps[ch + 2] = stage(ch + 2, b)
    for ch in (NCHA - 2, NCHA - 1):
        for c in out_cps.pop(ch):
            c.wait()

    mm_v[pl.ds(0, 16)] = mn
    mm_v[pl.ds(16, 16)] = mx
    pltpu.sync_copy(mm_v.at[pl.ds(0, 16)], mm_out.at[pl.ds(wid * 16, 16)])
    pltpu.sync_copy(mm_v.at[pl.ds(16, 16)],
                    mm_out.at[pl.ds(NW * 16 + wid * 16, 16)])


@functools.lru_cache(maxsize=1)
def _sc_a_call():
    return functools.partial(
        pl.kernel,
        out_type=[
            jax.ShapeDtypeStruct((N * K,), jnp.float32),
            jax.ShapeDtypeStruct((N * K,), jnp.float32),
            jax.ShapeDtypeStruct((2 * NW * 16,), jnp.float32),
        ],
        mesh=plsc.VectorSubcoreMesh(
            core_axis_name="c", subcore_axis_name="s",
            num_cores=NC, num_subcores=NS),
        compiler_params=pltpu.CompilerParams(
            needs_layout_passes=False, use_tc_tiling_on_sc=False),
        scratch_types=[
            pltpu.VMEM((2, CHA), jnp.int32),
            pltpu.VMEM((2, CHA * K), jnp.int32),
            pltpu.VMEM((2, CHA, K), jnp.float32),
            pltpu.VMEM((2, CHA, K), jnp.float32),
            pltpu.VMEM((2, CHA * K), jnp.float32),
            pltpu.VMEM((2, CHA * K), jnp.float32),
            pltpu.VMEM((2 * 16,), jnp.float32),
            pltpu.SemaphoreType.DMA((2,)),
            pltpu.SemaphoreType.DMA((2,)),
        ],
    )(_sc_a)


def _sc_b(dtf_hbm, atf_hbm, icf_hbm, mm_hbm, out_hbm,
          dt_v, at_v, ic_v, mm_v, buf_v, sems, semst):
    wid = lax.axis_index("s") * NC + lax.axis_index("c")
    base_w = wid * ROWS_W

    pltpu.sync_copy(mm_hbm, mm_v)

    def red_body(w, c):
        lo, hi = c
        lo = jnp.minimum(lo, mm_v[pl.ds(w * 16, 16)])
        hi = jnp.maximum(hi, mm_v[pl.ds(NW * 16 + w * 16, 16)])
        return lo, hi

    lo0 = jnp.full((16,), jnp.inf, jnp.float32)
    hi0 = jnp.full((16,), -jnp.inf, jnp.float32)
    lo, hi = lax.fori_loop(0, NW, red_body, (lo0, hi0))
    c22 = lax.broadcasted_iota(jnp.int32, (16,), 0) * 22
    ones = jnp.full((16,), 1.0, jnp.float32)
    zeros = jnp.zeros((16,), jnp.float32)
    gminv = jnp.full((16,), jnp.min(lo), jnp.float32)
    gmaxv = jnp.full((16,), jnp.max(hi), jnp.float32)
    invv = ones / (gmaxv - gminv)

    def z_body(i, _):
        for b in range(2):
            for t in range(24):
                buf_v[b, i, pl.ds(t * 16, 16)] = zeros
        return 0

    lax.fori_loop(0, PCB, z_body, 0)

    def stage(ch, sb):
        fbase = (base_w + ch * CH) * K
        return [
            pltpu.async_copy(dtf_hbm.at[pl.ds(fbase, CH * K)],
                             dt_v.at[sb], semst.at[sb]),
            pltpu.async_copy(atf_hbm.at[pl.ds(fbase, CH * K)],
                             at_v.at[sb], semst.at[sb]),
            pltpu.async_copy(icf_hbm.at[pl.ds(fbase, CH * K)],
                             ic_v.at[sb], semst.at[sb]),
        ]

    st_cps = {0: stage(0, 0)}
    for ch in range(NCH):
        sb = ch % 2
        base = base_w + ch * CH
        for c in st_cps.pop(ch):
            c.wait()
        if ch + 1 < NCH:
            st_cps[ch + 1] = stage(ch + 1, (ch + 1) % 2)

        def restore(p, b, sb=sb):
            def row_z(i, _):
                ic = p * PCB + i
                cls = ic_v[sb, pl.ds(ic * K, K)]
                rowi = jnp.full((16,), i, jnp.int32)
                plsc.store_scatter(buf_v.at[b], [rowi, c22 + cls], zeros)
                return 0

            lax.fori_loop(0, PCB, row_z, 0)

        def piece_body(p, _, base=base, sb=sb, restore=restore):
            b = lax.rem(p, 2)

            @pl.when(p >= 2)
            def _():
                pltpu.make_async_copy(
                    buf_v.at[b],
                    out_hbm.at[pl.ds(base + p * PCB, PCB), :],
                    sems.at[b]).wait()
                restore(p - 2, b)

            def row_w(i, _):
                ic = p * PCB + i
                fl = pl.ds(ic * K, K)
                cls = ic_v[sb, fl]
                rowi = jnp.full((16,), i, jnp.int32)
                plsc.store_scatter(buf_v.at[b], [rowi, c22 + cls], ones)
                dn = (dt_v[sb, fl] - gminv) * invv
                buf_v[b, i, pl.ds(352, 16)] = dn
                buf_v[b, i, pl.ds(368, 16)] = at_v[sb, fl]
                return 0

            lax.fori_loop(0, PCB, row_w, 0)
            pltpu.async_copy(
                buf_v.at[b], out_hbm.at[pl.ds(base + p * PCB, PCB), :],
                sems.at[b])
            return 0

        lax.fori_loop(0, NPIECE, piece_body, 0)
        for pp in (NPIECE - 2, NPIECE - 1):
            b = pp % 2
            pltpu.make_async_copy(
                buf_v.at[b],
                out_hbm.at[pl.ds(base + pp * PCB, PCB), :],
                sems.at[b]).wait()
            restore(pp, b)


@functools.lru_cache(maxsize=1)
def _sc_b_call():
    return functools.partial(
        pl.kernel,
        out_type=[
            jax.ShapeDtypeStruct((N, 384), jnp.float32),
        ],
        mesh=plsc.VectorSubcoreMesh(
            core_axis_name="c", subcore_axis_name="s",
            num_cores=NC, num_subcores=NS),
        compiler_params=pltpu.CompilerParams(
            needs_layout_passes=False, use_tc_tiling_on_sc=True),
        scratch_types=[
            pltpu.VMEM((2, CH * K), jnp.float32),
            pltpu.VMEM((2, CH * K), jnp.float32),
            pltpu.VMEM((2, CH * K), jnp.int32),
            pltpu.VMEM((2 * NW * 16,), jnp.float32),
            pltpu.VMEM((2, PCB, 384), jnp.float32),
            pltpu.SemaphoreType.DMA((2,)),
            pltpu.SemaphoreType.DMA((2,)),
        ],
    )(_sc_b)


def kernel(dist, angle, idx_t, index_t, index_h, device):
    del device
    idx_t = idx_t.astype(jnp.int32)
    index_t = index_t.astype(jnp.int32)
    index_h = index_h.astype(jnp.int32)

    idxt_f, icls_f = _sc_a0_call()(index_t, idx_t)
    dtf, atf, mm = _sc_a_call()(dist, angle, index_h, idxt_f)
    (out,) = _sc_b_call()(dtf, atf, icls_f, mm)
    return out


# speedup vs baseline: 1.1051x; 1.0388x over previous
import functools

import jax
import jax.numpy as jnp
from jax import lax
from jax.experimental import pallas as pl
from jax.experimental.pallas import tpu as pltpu
from jax.experimental.pallas import tpu_sc as plsc

N = 160000
K = 16
NC, NS = 2, 16
NW = NC * NS
ROWS_W = N // NW
CH = 1000
NCH = ROWS_W // CH
CHA = 200
NCHA = ROWS_W // CHA
_GSLICES = [(o, min(128, CHA - o)) for o in range(0, CHA, 128)]

PCB = 40
NPIECE = CH // PCB


CH0 = 200
NCH0 = ROWS_W // CH0


def _sc_a0(idxt_hbm, icls_hbm, idxtf_out, iclsf_out,
           t_v, c_v, tf_v, cf_v, sems, semo):
    wid = lax.axis_index("s") * NC + lax.axis_index("c")
    base_w = wid * ROWS_W

    def stage(ch, b):
        base = base_w + ch * CH0
        return [
            pltpu.async_copy(idxt_hbm.at[pl.ds(base, CH0), :],
                             t_v.at[b], sems.at[b]),
            pltpu.async_copy(icls_hbm.at[pl.ds(base, CH0), :],
                             c_v.at[b], sems.at[b]),
        ]

    in_cps = {0: stage(0, 0), 1: stage(1, 1)}
    out_cps = {}
    for ch in range(NCH0):
        b = ch % 2
        for c in in_cps.pop(ch):
            c.wait()
        if ch >= 2:
            for c in out_cps.pop(ch - 2):
                c.wait()

        def row_body(i, _, b=b):
            fl = pl.ds(i * K, K)
            tf_v[b, fl] = t_v[b, i, :]
            cf_v[b, fl] = c_v[b, i, :]
            return 0

        lax.fori_loop(0, CH0, row_body, 0)
        fbase = (base_w + ch * CH0) * K
        out_cps[ch] = [
            pltpu.async_copy(tf_v.at[b],
                             idxtf_out.at[pl.ds(fbase, CH0 * K)], semo.at[b]),
            pltpu.async_copy(cf_v.at[b],
                             iclsf_out.at[pl.ds(fbase, CH0 * K)], semo.at[b]),
        ]
        if ch + 2 < NCH0:
            in_cps[ch + 2] = stage(ch + 2, b)
    for ch in (NCH0 - 2, NCH0 - 1):
        for c in out_cps.pop(ch):
            c.wait()


@functools.lru_cache(maxsize=1)
def _sc_a0_call():
    return functools.partial(
        pl.kernel,
        out_type=[
            jax.ShapeDtypeStruct((N * K,), jnp.int32),
            jax.ShapeDtypeStruct((N * K,), jnp.int32),
        ],
        mesh=plsc.VectorSubcoreMesh(
            core_axis_name="c", subcore_axis_name="s",
            num_cores=NC, num_subcores=NS),
        compiler_params=pltpu.CompilerParams(
            needs_layout_passes=False, use_tc_tiling_on_sc=True),
        scratch_types=[
            pltpu.VMEM((2, CH0, K), jnp.int32),
            pltpu.VMEM((2, CH0, K), jnp.int32),
            pltpu.VMEM((2, CH0 * K), jnp.int32),
            pltpu.VMEM((2, CH0 * K), jnp.int32),
            pltpu.SemaphoreType.DMA((2,)),
            pltpu.SemaphoreType.DMA((2,)),
        ],
    )(_sc_a0)


def _sc_a(da_hbm, idxh_hbm, idxt_hbm,
          dtf_out, atf_out, mm_out,
          idx_v, cidx_v, darows_v,
          dout_v, aout_v, mm_v, sems, semo):
    wid = lax.axis_index("s") * NC + lax.axis_index("c")
    base_w = wid * ROWS_W

    def stage(ch, b):
        base = base_w + ch * CHA
        pltpu.sync_copy(idxh_hbm.at[pl.ds(base, CHA)], idx_v.at[b])
        cps = [
            pltpu.async_copy(idxt_hbm.at[pl.ds(base * K, CHA * K)],
                             cidx_v.at[b], sems.at[b]),
        ]
        for off, sz in _GSLICES:
            sl = pl.ds(off, sz)
            cps.append(pltpu.async_copy(
                da_hbm.at[idx_v.at[b].at[sl]],
                darows_v.at[b].at[sl, :], sems.at[b]))
        return cps

    def compute(b, carry):
        def row_body(i, rc):
            rmn, rmx = rc
            colraw = cidx_v[b, pl.ds(i * K, K)]
            col = jnp.minimum(colraw, K - 1)
            row = jnp.full((16,), i, jnp.int32)
            msk = colraw < K
            d = plsc.load_gather(darows_v.at[b], [row, col])
            a = plsc.load_gather(darows_v.at[b], [row, col + K])
            d = jnp.where(msk, d, 0.0)
            a = jnp.where(msk, a, 0.0)
            fl = pl.ds(i * K, K)
            dout_v[b, fl] = d
            aout_v[b, fl] = a
            return jnp.minimum(rmn, d), jnp.maximum(rmx, d)

        return lax.fori_loop(0, CHA, row_body, carry)

    mn = jnp.full((16,), jnp.inf, jnp.float32)
    mx = jnp.full((16,), -jnp.inf, jnp.float32)
    in_cps = {0: stage(0, 0), 1: stage(1, 1)}
    out_cps = {}
    for ch in range(NCHA):
        b = ch % 2
        for c in in_cps.pop(ch):
            c.wait()
        if ch >= 2:
            for c in out_cps.pop(ch - 2):
                c.wait()
        mn, mx = compute(b, (mn, mx))
        fbase = (base_w + ch * CHA) * K
        out_cps[ch] = [
            pltpu.async_copy(dout_v.at[b],
                             dtf_out.at[pl.ds(fbase, CHA * K)], semo.at[b]),
            pltpu.async_copy(aout_v.at[b],
                             atf_out.at[pl.ds(fbase, CHA * K)], semo.at[b]),
        ]
        if ch + 2 < NCHA:
            in_cps[ch + 2] = stage(ch + 2, b)
    for ch in (NCHA - 2, NCHA - 1):
        for c in out_cps.pop(ch):
            c.wait()

    mm_v[pl.ds(0, 16)] = mn
    mm_v[pl.ds(16, 16)] = mx
    pltpu.sync_copy(mm_v.at[pl.ds(0, 16)], mm_out.at[pl.ds(wid * 16, 16)])
    pltpu.sync_copy(mm_v.at[pl.ds(16, 16)],
                    mm_out.at[pl.ds(NW * 16 + wid * 16, 16)])


@functools.lru_cache(maxsize=1)
def _sc_a_call():
    return functools.partial(
        pl.kernel,
        out_type=[
            jax.ShapeDtypeStruct((N * K,), jnp.float32),
            jax.ShapeDtypeStruct((N * K,), jnp.float32),
            jax.ShapeDtypeStruct((2 * NW * 16,), jnp.float32),
        ],
        mesh=plsc.VectorSubcoreMesh(
            core_axis_name="c", subcore_axis_name="s",
            num_cores=NC, num_subcores=NS),
        compiler_params=pltpu.CompilerParams(
            needs_layout_passes=False, use_tc_tiling_on_sc=False),
        scratch_types=[
            pltpu.VMEM((2, CHA), jnp.int32),
            pltpu.VMEM((2, CHA * K), jnp.int32),
            pltpu.VMEM((2, CHA, 2 * K), jnp.float32),
            pltpu.VMEM((2, CHA * K), jnp.float32),
            pltpu.VMEM((2, CHA * K), jnp.float32),
            pltpu.VMEM((2 * 16,), jnp.float32),
            pltpu.SemaphoreType.DMA((2,)),
            pltpu.SemaphoreType.DMA((2,)),
        ],
    )(_sc_a)


def _sc_b(dtf_hbm, atf_hbm, icf_hbm, mm_hbm, out_hbm,
          dt_v, at_v, ic_v, mm_v, buf_v, sems, semst):
    wid = lax.axis_index("s") * NC + lax.axis_index("c")
    base_w = wid * ROWS_W

    pltpu.sync_copy(mm_hbm, mm_v)

    def red_body(w, c):
        lo, hi = c
        lo = jnp.minimum(lo, mm_v[pl.ds(w * 16, 16)])
        hi = jnp.maximum(hi, mm_v[pl.ds(NW * 16 + w * 16, 16)])
        return lo, hi

    lo0 = jnp.full((16,), jnp.inf, jnp.float32)
    hi0 = jnp.full((16,), -jnp.inf, jnp.float32)
    lo, hi = lax.fori_loop(0, NW, red_body, (lo0, hi0))
    c22 = lax.broadcasted_iota(jnp.int32, (16,), 0) * 22
    ones = jnp.full((16,), 1.0, jnp.float32)
    zeros = jnp.zeros((16,), jnp.float32)
    gminv = jnp.full((16,), jnp.min(lo), jnp.float32)
    gmaxv = jnp.full((16,), jnp.max(hi), jnp.float32)
    invv = ones / (gmaxv - gminv)

    def z_body(i, _):
        for b in range(2):
            for t in range(24):
                buf_v[b, i, pl.ds(t * 16, 16)] = zeros
        return 0

    lax.fori_loop(0, PCB, z_body, 0)

    def stage(ch, sb):
        fbase = (base_w + ch * CH) * K
        return [
            pltpu.async_copy(dtf_hbm.at[pl.ds(fbase, CH * K)],
                             dt_v.at[sb], semst.at[sb]),
            pltpu.async_copy(atf_hbm.at[pl.ds(fbase, CH * K)],
                             at_v.at[sb], semst.at[sb]),
            pltpu.async_copy(icf_hbm.at[pl.ds(fbase, CH * K)],
                             ic_v.at[sb], semst.at[sb]),
        ]

    st_cps = {0: stage(0, 0)}
    for ch in range(NCH):
        sb = ch % 2
        base = base_w + ch * CH
        for c in st_cps.pop(ch):
            c.wait()
        if ch + 1 < NCH:
            st_cps[ch + 1] = stage(ch + 1, (ch + 1) % 2)

        def restore(p, b, sb=sb):
            def row_z(i, _):
                ic = p * PCB + i
                cls = ic_v[sb, pl.ds(ic * K, K)]
                rowi = jnp.full((16,), i, jnp.int32)
                plsc.store_scatter(buf_v.at[b], [rowi, c22 + cls], zeros)
                return 0

            lax.fori_loop(0, PCB, row_z, 0)

        def piece_body(p, _, base=base, sb=sb, restore=restore):
            b = lax.rem(p, 2)

            @pl.when(p >= 2)
            def _():
                pltpu.make_async_copy(
                    buf_v.at[b],
                    out_hbm.at[pl.ds(base + p * PCB, PCB), :],
                    sems.at[b]).wait()
                restore(p - 2, b)

            def row_w(i, _):
                ic = p * PCB + i
                fl = pl.ds(ic * K, K)
                cls = ic_v[sb, fl]
                rowi = jnp.full((16,), i, jnp.int32)
                plsc.store_scatter(buf_v.at[b], [rowi, c22 + cls], ones)
                dn = (dt_v[sb, fl] - gminv) * invv
                buf_v[b, i, pl.ds(352, 16)] = dn
                buf_v[b, i, pl.ds(368, 16)] = at_v[sb, fl]
                return 0

            lax.fori_loop(0, PCB, row_w, 0)
            pltpu.async_copy(
                buf_v.at[b], out_hbm.at[pl.ds(base + p * PCB, PCB), :],
                sems.at[b])
            return 0

        lax.fori_loop(0, NPIECE, piece_body, 0)
        for pp in (NPIECE - 2, NPIECE - 1):
            b = pp % 2
            pltpu.make_async_copy(
                buf_v.at[b],
                out_hbm.at[pl.ds(base + pp * PCB, PCB), :],
                sems.at[b]).wait()
            restore(pp, b)


@functools.lru_cache(maxsize=1)
def _sc_b_call():
    return functools.partial(
        pl.kernel,
        out_type=[
            jax.ShapeDtypeStruct((N, 384), jnp.float32),
        ],
        mesh=plsc.VectorSubcoreMesh(
            core_axis_name="c", subcore_axis_name="s",
            num_cores=NC, num_subcores=NS),
        compiler_params=pltpu.CompilerParams(
            needs_layout_passes=False, use_tc_tiling_on_sc=True),
        scratch_types=[
            pltpu.VMEM((2, CH * K), jnp.float32),
            pltpu.VMEM((2, CH * K), jnp.float32),
            pltpu.VMEM((2, CH * K), jnp.int32),
            pltpu.VMEM((2 * NW * 16,), jnp.float32),
            pltpu.VMEM((2, PCB, 384), jnp.float32),
            pltpu.SemaphoreType.DMA((2,)),
            pltpu.SemaphoreType.DMA((2,)),
        ],
    )(_sc_b)


def kernel(dist, angle, idx_t, index_t, index_h, device):
    del device
    idx_t = idx_t.astype(jnp.int32)
    index_t = index_t.astype(jnp.int32)
    index_h = index_h.astype(jnp.int32)

    da = jnp.concatenate([dist, angle], axis=1)
    idxt_f, icls_f = _sc_a0_call()(index_t, idx_t)
    dtf, atf, mm = _sc_a_call()(da, index_h, idxt_f)
    (out,) = _sc_b_call()(dtf, atf, icls_f, mm)
    return out
